# baseline (device time: 154656 ns/iter reference)
import jax
import jax.numpy as jnp
from jax import lax
from jax.experimental import pallas as pl
from jax.experimental.pallas import tpu as pltpu

N_DEV = 4
SUB = 16

_DeviceIdType = getattr(pl, "DeviceIdType", None) or pltpu.DeviceIdType
_sem_signal = getattr(pl, "semaphore_signal", None) or pltpu.semaphore_signal
_sem_wait = getattr(pl, "semaphore_wait", None) or pltpu.semaphore_wait
_CompilerParams = getattr(pltpu, "CompilerParams", None) or getattr(
    pltpu, "TPUCompilerParams"
)
_ANY = getattr(pltpu, "ANY", None) or pl.ANY


def kernel(x):
    _, m, n = x.shape
    ch = n // N_DEV
    mh = m // 2
    qm = mh // SUB

    def body(
        x_hbm,
        out_ref,
        stage_ref,
        send0_ref,
        comm_cw,
        comm_ccw,
        local_sems,
        send_sems,
        recv_sems,
    ):
        p = lax.axis_index("i")
        left = lax.rem(p + N_DEV - 1, N_DEV)
        right = lax.rem(p + 1, N_DEV)
        comm = (comm_cw, comm_ccw)
        peer = (right, left)

        def col(c):
            return pl.ds(c * ch, ch)

        def rows(s):
            return pl.ds(s * qm, qm)

        def send_chunk(d, h):
            return lax.rem(p + 3 - h + N_DEV, N_DEV) if d == 0 else lax.rem(
                p + 1 + h, N_DEV
            )

        def recv_chunk(d, h):
            return lax.rem(p + 2 - h + N_DEV, N_DEV) if d == 0 else lax.rem(
                p + 2 + h, N_DEV
            )

        def load(d, s, c):
            return pltpu.make_async_copy(
                x_hbm.at[0, pl.ds(d * mh + s * qm, qm), col(c)],
                stage_ref.at[d, rows(s)],
                local_sems.at[d, s],
            )

        def make_rdma(d, h, s):
            src = send0_ref.at[d, rows(s)] if h == 0 else comm[d].at[h - 1, rows(s)]
            return pltpu.make_async_remote_copy(
                src_ref=src,
                dst_ref=comm[d].at[h, rows(s)],
                send_sem=send_sems.at[d, h, s],
                recv_sem=recv_sems.at[d, h, s],
                device_id=(peer[d],),
                device_id_type=_DeviceIdType.MESH,
            )

        rdmas = {}
        loads = {}

        for d in (0, 1):
            for s in range(SUB):
                loads[(d, -1, s)] = load(d, s, send_chunk(d, 0))
                loads[(d, -1, s)].start()

        barrier_sem = pltpu.get_barrier_semaphore()
        for nbr in (left, right):
            _sem_signal(
                barrier_sem,
                inc=1,
                device_id=(nbr,),
                device_id_type=_DeviceIdType.MESH,
            )
        _sem_wait(barrier_sem, 2)

        for s in range(SUB):
            for d in (0, 1):
                loads[(d, -1, s)].wait()
                send0_ref[d, rows(s)] = stage_ref[d, rows(s)].astype(jnp.bfloat16)
                rdmas[(d, 0, s)] = make_rdma(d, 0, s)
                rdmas[(d, 0, s)].start()
                loads[(d, 0, s)] = load(d, s, recv_chunk(d, 0))
                loads[(d, 0, s)].start()

        for h in range(N_DEV - 1):
            for s in range(SUB):
                for d in (0, 1):
                    rdmas[(d, h, s)].wait_recv()
                    loads[(d, h, s)].wait()
                    acc = comm[d][h, rows(s)] + stage_ref[d, rows(s)].astype(
                        jnp.bfloat16
                    )
                    if h < N_DEV - 2:
                        comm[d][h, rows(s)] = acc
                        rdmas[(d, h + 1, s)] = make_rdma(d, h + 1, s)
                        rdmas[(d, h + 1, s)].start()
                        loads[(d, h + 1, s)] = load(d, s, recv_chunk(d, h + 1))
                        loads[(d, h + 1, s)].start()
                    else:
                        out_ref[pl.ds(d * mh + s * qm, qm), :] = acc

        for key in rdmas:
            rdmas[key].wait_send()

    return pl.pallas_call(
        body,
        out_shape=jax.ShapeDtypeStruct((m, ch), jnp.bfloat16),
        in_specs=[pl.BlockSpec(memory_space=_ANY)],
        out_specs=pl.BlockSpec(memory_space=pltpu.VMEM),
        scratch_shapes=[
            pltpu.VMEM((2, mh, ch), jnp.float32),
            pltpu.VMEM((2, mh, ch), jnp.bfloat16),
            pltpu.VMEM((N_DEV - 1, mh, ch), jnp.bfloat16),
            pltpu.VMEM((N_DEV - 1, mh, ch), jnp.bfloat16),
            pltpu.SemaphoreType.DMA((2, SUB)),
            pltpu.SemaphoreType.DMA((2, N_DEV - 1, SUB)),
            pltpu.SemaphoreType.DMA((2, N_DEV - 1, SUB)),
        ],
        compiler_params=_CompilerParams(
            collective_id=0, vmem_limit_bytes=60 * 1024 * 1024
        ),
    )(x)


# device time: 153832 ns/iter; 1.0054x vs baseline; 1.0054x over previous
import jax
import jax.numpy as jnp
from jax import lax
from jax.experimental import pallas as pl
from jax.experimental.pallas import tpu as pltpu

N_DEV = 4
SUB = 8

_DeviceIdType = getattr(pl, "DeviceIdType", None) or pltpu.DeviceIdType
_sem_signal = getattr(pl, "semaphore_signal", None) or pltpu.semaphore_signal
_sem_wait = getattr(pl, "semaphore_wait", None) or pltpu.semaphore_wait
_CompilerParams = getattr(pltpu, "CompilerParams", None) or getattr(
    pltpu, "TPUCompilerParams"
)
_ANY = getattr(pltpu, "ANY", None) or pl.ANY


def kernel(x):
    _, m, n = x.shape
    ch = n // N_DEV
    mh = m // 2
    qm = mh // SUB

    def body(
        x_hbm,
        out_ref,
        stage_ref,
        send0_ref,
        comm_cw,
        comm_ccw,
        local_sems,
        send_sems,
        recv_sems,
    ):
        p = lax.axis_index("i")
        left = lax.rem(p + N_DEV - 1, N_DEV)
        right = lax.rem(p + 1, N_DEV)
        comm = (comm_cw, comm_ccw)
        peer = (right, left)

        def col(c):
            return pl.ds(c * ch, ch)

        def rows(s):
            return pl.ds(s * qm, qm)

        def send_chunk(d, h):
            return lax.rem(p + 3 - h + N_DEV, N_DEV) if d == 0 else lax.rem(
                p + 1 + h, N_DEV
            )

        def recv_chunk(d, h):
            return lax.rem(p + 2 - h + N_DEV, N_DEV) if d == 0 else lax.rem(
                p + 2 + h, N_DEV
            )

        def load(d, s, c):
            return pltpu.make_async_copy(
                x_hbm.at[0, pl.ds(d * mh + s * qm, qm), col(c)],
                stage_ref.at[d, rows(s)],
                local_sems.at[d, s],
            )

        def make_rdma(d, h, s):
            src = send0_ref.at[d, rows(s)] if h == 0 else comm[d].at[h - 1, rows(s)]
            return pltpu.make_async_remote_copy(
                src_ref=src,
                dst_ref=comm[d].at[h, rows(s)],
                send_sem=send_sems.at[d, h, s],
                recv_sem=recv_sems.at[d, h, s],
                device_id=(peer[d],),
                device_id_type=_DeviceIdType.MESH,
            )

        rdmas = {}
        loads = {}

        for d in (0, 1):
            for s in range(SUB):
                loads[(d, -1, s)] = load(d, s, send_chunk(d, 0))
                loads[(d, -1, s)].start()

        barrier_sem = pltpu.get_barrier_semaphore()
        for nbr in (left, right):
            _sem_signal(
                barrier_sem,
                inc=1,
                device_id=(nbr,),
                device_id_type=_DeviceIdType.MESH,
            )
        _sem_wait(barrier_sem, 2)

        for s in range(SUB):
            for d in (0, 1):
                loads[(d, -1, s)].wait()
                send0_ref[d, rows(s)] = stage_ref[d, rows(s)].astype(jnp.bfloat16)
                rdmas[(d, 0, s)] = make_rdma(d, 0, s)
                rdmas[(d, 0, s)].start()
                loads[(d, 0, s)] = load(d, s, recv_chunk(d, 0))
                loads[(d, 0, s)].start()

        for h in range(N_DEV - 1):
            for s in range(SUB):
                for d in (0, 1):
                    rdmas[(d, h, s)].wait_recv()
                    loads[(d, h, s)].wait()
                    acc = comm[d][h, rows(s)] + stage_ref[d, rows(s)].astype(
                        jnp.bfloat16
                    )
                    if h < N_DEV - 2:
                        comm[d][h, rows(s)] = acc
                        rdmas[(d, h + 1, s)] = make_rdma(d, h + 1, s)
                        rdmas[(d, h + 1, s)].start()
                        loads[(d, h + 1, s)] = load(d, s, recv_chunk(d, h + 1))
                        loads[(d, h + 1, s)].start()
                    else:
                        out_ref[pl.ds(d * mh + s * qm, qm), :] = acc

        for key in rdmas:
            rdmas[key].wait_send()

    return pl.pallas_call(
        body,
        out_shape=jax.ShapeDtypeStruct((m, ch), jnp.bfloat16),
        in_specs=[pl.BlockSpec(memory_space=_ANY)],
        out_specs=pl.BlockSpec(memory_space=pltpu.VMEM),
        scratch_shapes=[
            pltpu.VMEM((2, mh, ch), jnp.float32),
            pltpu.VMEM((2, mh, ch), jnp.bfloat16),
            pltpu.VMEM((N_DEV - 1, mh, ch), jnp.bfloat16),
            pltpu.VMEM((N_DEV - 1, mh, ch), jnp.bfloat16),
            pltpu.SemaphoreType.DMA((2, SUB)),
            pltpu.SemaphoreType.DMA((2, N_DEV - 1, SUB)),
            pltpu.SemaphoreType.DMA((2, N_DEV - 1, SUB)),
        ],
        compiler_params=_CompilerParams(
            collective_id=0, vmem_limit_bytes=60 * 1024 * 1024
        ),
    )(x)
